# baseline (speedup 1.0000x reference)
import jax
import jax.numpy as jnp
from jax.experimental import pallas as pl
from jax.experimental.pallas import tpu as pltpu

_T, _N, _D, _M = 2, 100, 128, 1024
_NS, _NQ = 25, 75
_R = _T * _N
_MARGIN = 0.5


def _l2n(x):
    return x / jnp.maximum(jnp.sqrt(jnp.sum(x * x, axis=-1, keepdims=True)), 1e-12)


def _body(es_ref, eq_ref, gs_ref, gq_ref, th_ref, k_ref, v_ref,
          nemb_ref, eg_ref, lk_ref, lv_ref, ls_ref):
    kmat = k_ref[...]
    vmat = v_ref[...]
    th0 = th_ref[0]
    th1 = th_ref[1]
    th2 = th_ref[2]
    th3 = th_ref[3]
    big = jnp.int32(2 ** 30)
    inf = jnp.float32(jnp.inf)

    def pair_contrib(src, other, thp, thn, iota):
        mp = jnp.where(src > thp, src, inf)
        mn = jnp.where(src < thn, src, -inf)
        extp = jnp.min(mp, axis=1, keepdims=True)
        extn = jnp.max(mn, axis=1, keepdims=True)
        idxp = jnp.min(jnp.where(mp == extp, iota, big), axis=1, keepdims=True)
        idxn = jnp.min(jnp.where(mn == extn, iota, big), axis=1, keepdims=True)
        anyp = (extp != inf).astype(jnp.float32)
        anyn = (extn != -inf).astype(jnp.float32)
        w = (iota == idxp).astype(jnp.float32) * anyp \
            - (iota == idxn).astype(jnp.float32) * anyn
        return jnp.sum(w * other)

    def part(e, g):
        r = e.shape[0]
        ne = _l2n(e)
        ng = _l2n(g)
        sim_kv = jax.lax.dot_general(ne, kmat, (((1,), (1,)), ((), ())),
                                     preferred_element_type=jnp.float32)
        sim_vk = jax.lax.dot_general(ng, vmat, (((1,), (1,)), ((), ())),
                                     preferred_element_type=jnp.float32)
        pos_score = jnp.where(sim_kv > th0, sim_kv, 0.0)
        eg = _l2n(ng + jax.lax.dot_general(
            pos_score, vmat, (((1,), (0,)), ((), ())),
            preferred_element_type=jnp.float32))
        diff = sim_vk - sim_kv
        ls_p = jnp.sum(diff * diff)
        iota = jax.lax.broadcasted_iota(jnp.int32, (r, _M), 1)
        lv_p = pair_contrib(sim_kv, sim_vk, th0, th1, iota)
        lk_p = pair_contrib(sim_vk, sim_kv, th2, th3, iota)
        return ne, eg, lk_p, lv_p, ls_p

    ne_s, eg_s, lk_s, lv_s, ls_s = part(es_ref[...], gs_ref[...])
    ne_q, eg_q, lk_q, lv_q, ls_q = part(eq_ref[...], gq_ref[...])

    nemb_ref[0:_NS, :] = ne_s[0:_NS]
    nemb_ref[_NS:_N, :] = ne_q[0:_NQ]
    nemb_ref[_N:_N + _NS, :] = ne_s[_NS:2 * _NS]
    nemb_ref[_N + _NS:_R, :] = ne_q[_NQ:2 * _NQ]
    eg_ref[0:_NS, :] = eg_s[0:_NS]
    eg_ref[_NS:_N, :] = eg_q[0:_NQ]
    eg_ref[_N:_N + _NS, :] = eg_s[_NS:2 * _NS]
    eg_ref[_N + _NS:_R, :] = eg_q[_NQ:2 * _NQ]

    ls_ref[...] = ((ls_s + ls_q) / (_R * _M)).reshape(1, 1)
    lv_ref[...] = jnp.maximum(-(lv_s + lv_q) / _R + _MARGIN, 0.0).reshape(1, 1)
    lk_ref[...] = jnp.maximum(-(lk_s + lk_q) / _R + _MARGIN, 0.0).reshape(1, 1)


def kernel(emb_support, emb_query, glo_support, glo_query, thresh,
           memory_keys, memory_values):
    es = emb_support.reshape(_T * _NS, _D)
    eq = emb_query.reshape(_T * _NQ, _D)
    gs = glo_support.reshape(_T * _NS, _D)
    gq = glo_query.reshape(_T * _NQ, _D)

    out_shape = (
        jax.ShapeDtypeStruct((_R, _D), jnp.float32),
        jax.ShapeDtypeStruct((_R, _D), jnp.float32),
        jax.ShapeDtypeStruct((1, 1), jnp.float32),
        jax.ShapeDtypeStruct((1, 1), jnp.float32),
        jax.ShapeDtypeStruct((1, 1), jnp.float32),
    )
    vspec = pl.BlockSpec(memory_space=pltpu.VMEM)
    in_specs = [vspec, vspec, vspec, vspec,
                pl.BlockSpec(memory_space=pltpu.SMEM), vspec, vspec]
    out_specs = (vspec,) * 5
    ne, eg, lk, lv, ls = pl.pallas_call(
        _body,
        out_shape=out_shape,
        in_specs=in_specs,
        out_specs=out_specs,
    )(es, eq, gs, gq, thresh, memory_keys, memory_values)

    return (ne.reshape(_T, _N, _D), eg.reshape(_T, _N, _D),
            lk[0, 0], lv[0, 0], ls[0, 0])


# speedup vs baseline: 1.0416x; 1.0416x over previous
import jax
import jax.numpy as jnp
from jax.experimental import pallas as pl
from jax.experimental.pallas import tpu as pltpu

_T, _N, _D, _M = 2, 100, 128, 1024
_NS, _NQ = 25, 75
_R = _T * _N
_MARGIN = 0.5


def _l2n(x):
    return x / jnp.maximum(jnp.sqrt(jnp.sum(x * x, axis=-1, keepdims=True)), 1e-12)


def _store_tmajor(ref, x):
    ref[0:_NS, :] = x[0:_NS]
    ref[_NS:_N, :] = x[2 * _NS:2 * _NS + _NQ]
    ref[_N:_N + _NS, :] = x[_NS:2 * _NS]
    ref[_N + _NS:_R, :] = x[2 * _NS + _NQ:_R]


def _body(es_ref, eq_ref, gs_ref, gq_ref, th_ref, k_ref, v_ref,
          nemb_ref, eg_ref, lk_ref, lv_ref, ls_ref):
    ne = _l2n(jnp.concatenate([es_ref[...], eq_ref[...]], axis=0))
    ng = _l2n(jnp.concatenate([gs_ref[...], gq_ref[...]], axis=0))
    _store_tmajor(nemb_ref, ne)

    kmat = k_ref[...]
    vmat = v_ref[...]
    sim_kv = jax.lax.dot_general(ne, kmat, (((1,), (1,)), ((), ())),
                                 preferred_element_type=jnp.float32)
    sim_vk = jax.lax.dot_general(ng, vmat, (((1,), (1,)), ((), ())),
                                 preferred_element_type=jnp.float32)

    th0 = th_ref[0]
    th1 = th_ref[1]
    th2 = th_ref[2]
    th3 = th_ref[3]

    pos_score = jnp.where(sim_kv > th0, sim_kv, 0.0)
    eg = ng + jax.lax.dot_general(pos_score, vmat, (((1,), (0,)), ((), ())),
                                  preferred_element_type=jnp.float32)
    _store_tmajor(eg_ref, _l2n(eg))

    diff = sim_vk - sim_kv
    ls_ref[...] = jnp.sum(diff * diff, keepdims=True).reshape(1, 1) / (_R * _M)

    iota = jax.lax.broadcasted_iota(jnp.int32, (_R, _M), 1)
    big = jnp.int32(2 ** 30)
    inf = jnp.float32(jnp.inf)

    def pair_contrib(src, other, thp, thn):
        mp = jnp.where(src > thp, src, inf)
        mn = jnp.where(src < thn, src, -inf)
        extp = jnp.min(mp, axis=1, keepdims=True)
        extn = jnp.max(mn, axis=1, keepdims=True)
        idxp = jnp.min(jnp.where(mp == extp, iota, big), axis=1, keepdims=True)
        idxn = jnp.min(jnp.where(mn == extn, iota, big), axis=1, keepdims=True)
        valp = jnp.sum(jnp.where(iota == idxp, other, 0.0), axis=1, keepdims=True)
        valn = jnp.sum(jnp.where(iota == idxn, other, 0.0), axis=1, keepdims=True)
        anyp = (extp != inf).astype(jnp.float32)
        anyn = (extn != -inf).astype(jnp.float32)
        return jnp.sum(anyp * valp - anyn * valn, keepdims=True).reshape(1, 1)

    lv_ref[...] = jnp.maximum(
        -pair_contrib(sim_kv, sim_vk, th0, th1) / _R + _MARGIN, 0.0)
    lk_ref[...] = jnp.maximum(
        -pair_contrib(sim_vk, sim_kv, th2, th3) / _R + _MARGIN, 0.0)


def kernel(emb_support, emb_query, glo_support, glo_query, thresh,
           memory_keys, memory_values):
    es = emb_support.reshape(_T * _NS, _D)
    eq = emb_query.reshape(_T * _NQ, _D)
    gs = glo_support.reshape(_T * _NS, _D)
    gq = glo_query.reshape(_T * _NQ, _D)

    out_shape = (
        jax.ShapeDtypeStruct((_R, _D), jnp.float32),
        jax.ShapeDtypeStruct((_R, _D), jnp.float32),
        jax.ShapeDtypeStruct((1, 1), jnp.float32),
        jax.ShapeDtypeStruct((1, 1), jnp.float32),
        jax.ShapeDtypeStruct((1, 1), jnp.float32),
    )
    vspec = pl.BlockSpec(memory_space=pltpu.VMEM)
    in_specs = [vspec, vspec, vspec, vspec,
                pl.BlockSpec(memory_space=pltpu.SMEM), vspec, vspec]
    out_specs = (vspec,) * 5
    ne, eg, lk, lv, ls = pl.pallas_call(
        _body,
        out_shape=out_shape,
        in_specs=in_specs,
        out_specs=out_specs,
    )(es, eq, gs, gq, thresh, memory_keys, memory_values)

    return (ne.reshape(_T, _N, _D), eg.reshape(_T, _N, _D),
            lk[0, 0], lv[0, 0], ls[0, 0])


# speedup vs baseline: 1.1416x; 1.0960x over previous
import jax
import jax.numpy as jnp
from jax.experimental import pallas as pl
from jax.experimental.pallas import tpu as pltpu

_T, _N, _D, _M = 2, 100, 128, 1024
_R = _T * _N
_MARGIN = 0.5


def _l2n(x):
    return x / jnp.maximum(jnp.sqrt(jnp.sum(x * x, axis=-1, keepdims=True)), 1e-12)


def _body(emb_ref, glo_ref, th_ref, k_ref, v_ref,
          nemb_ref, eg_ref, lk_ref, lv_ref, ls_ref):
    ne = _l2n(emb_ref[...])
    ng = _l2n(glo_ref[...])
    nemb_ref[...] = ne

    kmat = k_ref[...]
    vmat = v_ref[...]
    sim_kv = jax.lax.dot_general(ne, kmat, (((1,), (1,)), ((), ())),
                                 preferred_element_type=jnp.float32)
    sim_vk = jax.lax.dot_general(ng, vmat, (((1,), (1,)), ((), ())),
                                 preferred_element_type=jnp.float32)

    th0 = th_ref[0]
    th1 = th_ref[1]
    th2 = th_ref[2]
    th3 = th_ref[3]

    pos_score = jnp.where(sim_kv > th0, sim_kv, 0.0)
    eg = ng + jax.lax.dot_general(pos_score, vmat, (((1,), (0,)), ((), ())),
                                  preferred_element_type=jnp.float32)
    eg_ref[...] = _l2n(eg)

    diff = sim_vk - sim_kv
    ls_ref[0] = jnp.sum(diff * diff) / (_R * _M)

    iota = jax.lax.broadcasted_iota(jnp.int32, (_R, _M), 1)
    big = jnp.int32(2 ** 30)
    inf = jnp.float32(jnp.inf)

    def pair_contrib(src, other, thp, thn):
        mp = jnp.where(src > thp, src, inf)
        mn = jnp.where(src < thn, src, -inf)
        extp = jnp.min(mp, axis=1, keepdims=True)
        extn = jnp.max(mn, axis=1, keepdims=True)
        idxp = jnp.min(jnp.where(mp == extp, iota, big), axis=1, keepdims=True)
        idxn = jnp.min(jnp.where(mn == extn, iota, big), axis=1, keepdims=True)
        valp = jnp.sum(jnp.where(iota == idxp, other, 0.0), axis=1, keepdims=True)
        valn = jnp.sum(jnp.where(iota == idxn, other, 0.0), axis=1, keepdims=True)
        anyp = (extp != inf).astype(jnp.float32)
        anyn = (extn != -inf).astype(jnp.float32)
        return jnp.sum(anyp * valp - anyn * valn)

    lv_ref[0] = jnp.maximum(
        -pair_contrib(sim_kv, sim_vk, th0, th1) / _R + _MARGIN, 0.0)
    lk_ref[0] = jnp.maximum(
        -pair_contrib(sim_vk, sim_kv, th2, th3) / _R + _MARGIN, 0.0)


def kernel(emb_support, emb_query, glo_support, glo_query, thresh,
           memory_keys, memory_values):
    emb = jnp.concatenate([emb_support, emb_query], axis=1).reshape(_R, _D)
    glo = jnp.concatenate([glo_support, glo_query], axis=1).reshape(_R, _D)

    out_shape = (
        jax.ShapeDtypeStruct((_R, _D), jnp.float32),
        jax.ShapeDtypeStruct((_R, _D), jnp.float32),
        jax.ShapeDtypeStruct((1,), jnp.float32),
        jax.ShapeDtypeStruct((1,), jnp.float32),
        jax.ShapeDtypeStruct((1,), jnp.float32),
    )
    vspec = pl.BlockSpec(memory_space=pltpu.VMEM)
    sspec = pl.BlockSpec(memory_space=pltpu.SMEM)
    in_specs = [vspec, vspec, sspec, vspec, vspec]
    out_specs = (vspec, vspec, sspec, sspec, sspec)
    ne, eg, lk, lv, ls = pl.pallas_call(
        _body,
        out_shape=out_shape,
        in_specs=in_specs,
        out_specs=out_specs,
    )(emb, glo, thresh, memory_keys, memory_values)

    return (ne.reshape(_T, _N, _D), eg.reshape(_T, _N, _D),
            lk[0], lv[0], ls[0])
